# Initial kernel scaffold; baseline (speedup 1.0000x reference)
import jax
import jax.numpy as jnp
from jax.experimental import pallas as pl


def kernel(feature_list, adj_list, alignment_list, train_year, bases0, coeffs0, wself0, bases1, coeffs1, wself1):
    raise NotImplementedError("write your pallas kernel here")


# speedup vs baseline: 8.3979x; 8.3979x over previous
import functools

import jax
import jax.numpy as jnp
from jax import lax
from jax.experimental import pallas as pl
from jax.experimental.pallas import tpu as pltpu
from jax.experimental.pallas import tpu_sc as plsc

N = 10000
E = 320000
D = 128
R = 4
NB = 4
T = 2

NW = 16
NPAD = 10240
RPT = NPAD // NW
EPW = 20480
CH = 64
IBLK = 32
NIB = EPW // (IBLK * CH)
DCH = 128
DIBLK = 32
DNIB = EPW // (DIBLK * DCH)
BN = 1000
GRID = N // BN

_f32 = jnp.float32

_sc_mesh = plsc.VectorSubcoreMesh(core_axis_name="c", subcore_axis_name="s",
                                  num_cores=1)


@functools.partial(
    pl.kernel,
    out_type=jax.ShapeDtypeStruct((NPAD, D), _f32),
    mesh=_sc_mesh,
    scratch_types=[
        pltpu.VMEM((IBLK, CH), jnp.int32),
        pltpu.VMEM((IBLK, CH), jnp.int32),
        pltpu.VMEM((CH, D), _f32),
        pltpu.VMEM((CH, D), _f32),
        pltpu.VMEM_SHARED((NPAD, D), _f32),
        pltpu.SemaphoreType.DMA,
        pltpu.SemaphoreType.DMA,
    ],
)
def _sc_agg(table, gidx_h, dst_h, z_h,
            acc_out,
            gidx_v, dst_v, bufa, bufb, acc_sh, sem_a, sem_b):
    wid = lax.axis_index("s")

    pltpu.sync_copy(z_h, bufa)
    base = wid * RPT

    def _zero(i, carry):
        pltpu.sync_copy(bufa, acc_sh.at[pl.ds(base + i * CH, CH)])
        return carry

    lax.fori_loop(0, RPT // CH, _zero, 0)
    plsc.subcore_barrier()

    def _block(b, carry):
        pltpu.sync_copy(gidx_h.at[wid, b], gidx_v)
        pltpu.sync_copy(dst_h.at[wid, b], dst_v)
        pltpu.async_copy(table.at[gidx_v.at[0]], bufa, sem_a)

        def _pair(jj, c):
            j0 = 2 * jj
            pltpu.make_async_copy(table.at[gidx_v.at[j0]], bufa,
                                  sem_a).wait()
            pltpu.async_copy(table.at[gidx_v.at[j0 + 1]], bufb, sem_b)
            pltpu.sync_copy(bufa, acc_sh.at[dst_v.at[j0]], add=True)
            pltpu.make_async_copy(table.at[gidx_v.at[j0 + 1]], bufb,
                                  sem_b).wait()

            @pl.when(jj < IBLK // 2 - 1)
            def _prefetch():
                pltpu.async_copy(table.at[gidx_v.at[j0 + 2]], bufa, sem_a)

            pltpu.sync_copy(bufb, acc_sh.at[dst_v.at[j0 + 1]], add=True)
            return c

        lax.fori_loop(0, IBLK // 2, _pair, 0)
        return carry

    lax.fori_loop(0, NIB, _block, 0)
    plsc.subcore_barrier()

    pltpu.sync_copy(acc_sh.at[pl.ds(base, RPT)],
                    acc_out.at[pl.ds(base, RPT)])


DBLK = DIBLK * DCH


@functools.partial(
    pl.kernel,
    out_type=jax.ShapeDtypeStruct((NW, NPAD), _f32),
    mesh=_sc_mesh,
    compiler_params=pltpu.CompilerParams(needs_layout_passes=False),
    scratch_types=[
        pltpu.VMEM((DBLK,), jnp.int32),
        pltpu.VMEM((NPAD,), _f32),
    ],
)
def _sc_deg(dst_h, zvec_h, deg_out, dst_v, deg_local):
    wid = lax.axis_index("s")
    pltpu.sync_copy(zvec_h, deg_local)
    ones16 = jnp.ones((16,), _f32)

    def _block(b, carry):
        pltpu.sync_copy(dst_h.at[wid, b], dst_v)

        def _step(i, c):
            idx = dst_v[pl.ds(i * 16, 16)]
            plsc.addupdate_scatter(deg_local, [idx], ones16)
            return c

        lax.fori_loop(0, DBLK // 16, _step, 0)
        return carry

    lax.fori_loop(0, DNIB, _block, 0)
    pltpu.sync_copy(deg_local, deg_out.at[wid])


def _wcomb_body(c0_ref, b0_ref, c1_ref, b1_ref, w0_ref, w1_ref):
    w0_ref[...] = jnp.dot(c0_ref[...], b0_ref[...],
                          preferred_element_type=_f32)
    w1_ref[...] = jnp.dot(c1_ref[...], b1_ref[...],
                          preferred_element_type=_f32)


def _combine_weights(coeffs0, bases0, coeffs1, bases1):
    w0, w1 = pl.pallas_call(
        _wcomb_body,
        out_shape=(jax.ShapeDtypeStruct((R, D * D), _f32),
                   jax.ShapeDtypeStruct((R, D * D), _f32)),
    )(coeffs0, bases0.reshape(NB, D * D), coeffs1, bases1.reshape(NB, D * D))
    return w0.reshape(R, D, D), w1.reshape(R, D, D)


def _prep_body(x_ref, w_ref, wself_ref, xw_ref, self_ref):
    x = x_ref[...]
    for r in range(R):
        xw_ref[r] = jnp.dot(x, w_ref[r], preferred_element_type=_f32)
    self_ref[...] = jnp.dot(x, wself_ref[...], preferred_element_type=_f32)


def _prep(x, w, wself):
    return pl.pallas_call(
        _prep_body,
        grid=(GRID,),
        in_specs=[
            pl.BlockSpec((BN, D), lambda i: (i, 0)),
            pl.BlockSpec((R, D, D), lambda i: (0, 0, 0)),
            pl.BlockSpec((D, D), lambda i: (0, 0)),
        ],
        out_specs=(
            pl.BlockSpec((R, BN, D), lambda i: (0, i, 0)),
            pl.BlockSpec((BN, D), lambda i: (i, 0)),
        ),
        out_shape=(jax.ShapeDtypeStruct((R, N, D), _f32),
                   jax.ShapeDtypeStruct((N, D), _f32)),
    )(x, w, wself)


def _mid_body(acc_ref, deg_ref, self_ref, w_ref, wself_ref,
              xw_ref, self1_ref):
    deg = jnp.sum(deg_ref[...], axis=1, keepdims=True)
    degc = jnp.maximum(deg, 1.0)
    x1 = acc_ref[...] / degc + self_ref[...]
    x1 = jnp.maximum(x1, 0.0)
    for r in range(R):
        xw_ref[r] = jnp.dot(x1, w_ref[r], preferred_element_type=_f32)
    self1_ref[...] = jnp.dot(x1, wself_ref[...], preferred_element_type=_f32)


def _mid(acc, degcol, self0, w, wself):
    return pl.pallas_call(
        _mid_body,
        grid=(GRID,),
        in_specs=[
            pl.BlockSpec((BN, D), lambda i: (i, 0)),
            pl.BlockSpec((BN, NW), lambda i: (i, 0)),
            pl.BlockSpec((BN, D), lambda i: (i, 0)),
            pl.BlockSpec((R, D, D), lambda i: (0, 0, 0)),
            pl.BlockSpec((D, D), lambda i: (0, 0)),
        ],
        out_specs=(
            pl.BlockSpec((R, BN, D), lambda i: (0, i, 0)),
            pl.BlockSpec((BN, D), lambda i: (i, 0)),
        ),
        out_shape=(jax.ShapeDtypeStruct((R, N, D), _f32),
                   jax.ShapeDtypeStruct((N, D), _f32)),
    )(acc, degcol, self0, w, wself)


def _post_body(acc_ref, deg_ref, self_ref, out_ref):
    deg = jnp.sum(deg_ref[...], axis=1, keepdims=True)
    degc = jnp.maximum(deg, 1.0)
    out_ref[...] = acc_ref[...] / degc + self_ref[...]


def _post(acc, degcol, self1):
    return pl.pallas_call(
        _post_body,
        grid=(GRID,),
        in_specs=[
            pl.BlockSpec((BN, D), lambda i: (i, 0)),
            pl.BlockSpec((BN, NW), lambda i: (i, 0)),
            pl.BlockSpec((BN, D), lambda i: (i, 0)),
        ],
        out_specs=pl.BlockSpec((BN, D), lambda i: (i, 0)),
        out_shape=jax.ShapeDtypeStruct((N, D), _f32),
    )(acc, degcol, self1)


def kernel(feature_list, adj_list, alignment_list, train_year,
           bases0, coeffs0, wself0, bases1, coeffs1, wself1):
    del alignment_list
    w0, w1 = _combine_weights(coeffs0, bases0, coeffs1, bases1)

    z64 = jnp.zeros((CH, D), _f32)
    zvec = jnp.zeros((NPAD,), _f32)

    pad = NW * EPW - E
    outs = []
    prev = None
    for t in range(T):
        x = feature_list[t]
        edges = adj_list[t].astype(jnp.int32)
        src, dst = edges[0], edges[1]
        rel = edges[2] % R
        gidx = rel * N + src
        gidxp = jnp.concatenate([gidx, jnp.zeros((pad,), jnp.int32)])
        dstp = jnp.concatenate([dst, jnp.full((pad,), N, jnp.int32)])
        gidx4 = gidxp.reshape(NW, NIB, IBLK, CH)
        dst4 = dstp.reshape(NW, NIB, IBLK, CH)
        dstd = dstp.reshape(NW, DNIB, DBLK)
        if prev is not None:
            gidx4, dst4, dstd, _ = lax.optimization_barrier(
                (gidx4, dst4, dstd, prev))

        deg = _sc_deg(dstd, zvec)
        degt = deg[:, :N].T
        xw0, s0 = _prep(x, w0, wself0)
        gidx4b, dst4b, _ = lax.optimization_barrier((gidx4, dst4, deg))
        acc0 = _sc_agg(xw0.reshape(R * N, D), gidx4b, dst4b, z64)
        xw1, s1 = _mid(acc0, degt, s0, w1, wself1)
        acc1 = _sc_agg(xw1.reshape(R * N, D), gidx4b, dst4b, z64)
        outs.append(_post(acc1, degt, s1))
        prev = acc1

    emb = jnp.stack(outs)
    mask = (jnp.arange(T) < train_year).astype(emb.dtype)
    return emb * mask[:, None, None]


# speedup vs baseline: 8.9282x; 1.0632x over previous
import functools

import jax
import jax.numpy as jnp
from jax import lax
from jax.experimental import pallas as pl
from jax.experimental.pallas import tpu as pltpu
from jax.experimental.pallas import tpu_sc as plsc

N = 10000
E = 320000
D = 128
R = 4
NB = 4
T = 2

NW = 16
NPAD = 10240
RPT = NPAD // NW
EPW = 20480
CH = 32
NSLOT = 4
IBLK = 64
NIB = EPW // (IBLK * CH)
DCH = 128
DIBLK = 32
DNIB = EPW // (DIBLK * DCH)
BN = 1000
GRID = N // BN

_f32 = jnp.float32

_sc_mesh = plsc.VectorSubcoreMesh(core_axis_name="c", subcore_axis_name="s",
                                  num_cores=1)


@functools.partial(
    pl.kernel,
    out_type=jax.ShapeDtypeStruct((NPAD, D), _f32),
    mesh=_sc_mesh,
    scratch_types=[
        pltpu.VMEM((IBLK, CH), jnp.int32),
        pltpu.VMEM((IBLK, CH), jnp.int32),
        [pltpu.VMEM((CH, D), _f32)] * NSLOT,
        [pltpu.SemaphoreType.DMA] * NSLOT,
        [pltpu.SemaphoreType.DMA] * NSLOT,
        pltpu.VMEM_SHARED((NPAD, D), _f32),
    ],
)
def _sc_agg(table, gidx_h, dst_h, z_h,
            acc_out,
            gidx_v, dst_v, bufs, sem_g, sem_c, acc_sh):
    wid = lax.axis_index("s")

    pltpu.sync_copy(z_h, bufs[0])
    base = wid * RPT

    def _zero(i, carry):
        pltpu.sync_copy(bufs[0], acc_sh.at[pl.ds(base + i * CH, CH)])
        return carry

    lax.fori_loop(0, RPT // CH, _zero, 0)
    plsc.subcore_barrier()

    def _block(b, carry):
        pltpu.sync_copy(gidx_h.at[wid, b], gidx_v)
        pltpu.sync_copy(dst_h.at[wid, b], dst_v)
        for s in range(NSLOT):
            pltpu.async_copy(table.at[gidx_v.at[s]], bufs[s], sem_g[s])

        def _group(g, c):
            j0 = g * NSLOT
            for s in range(NSLOT):
                pltpu.make_async_copy(table.at[gidx_v.at[j0 + s]], bufs[s],
                                      sem_g[s]).wait()
                pltpu.async_copy(bufs[s], acc_sh.at[dst_v.at[j0 + s]],
                                 sem_c[s], add=True)
            for s in range(NSLOT):
                pltpu.make_async_copy(bufs[s], acc_sh.at[dst_v.at[j0 + s]],
                                      sem_c[s]).wait()

                @pl.when(g < IBLK // NSLOT - 1)
                def _prefetch(s=s, j0=j0):
                    pltpu.async_copy(table.at[gidx_v.at[j0 + NSLOT + s]],
                                     bufs[s], sem_g[s])

            return c

        lax.fori_loop(0, IBLK // NSLOT, _group, 0)
        return carry

    lax.fori_loop(0, NIB, _block, 0)
    plsc.subcore_barrier()

    pltpu.sync_copy(acc_sh.at[pl.ds(base, RPT)],
                    acc_out.at[pl.ds(base, RPT)])


DBLK = DIBLK * DCH


@functools.partial(
    pl.kernel,
    out_type=jax.ShapeDtypeStruct((NW, NPAD), _f32),
    mesh=_sc_mesh,
    compiler_params=pltpu.CompilerParams(needs_layout_passes=False),
    scratch_types=[
        pltpu.VMEM((DBLK,), jnp.int32),
        pltpu.VMEM((NPAD,), _f32),
    ],
)
def _sc_deg(dst_h, zvec_h, deg_out, dst_v, deg_local):
    wid = lax.axis_index("s")
    pltpu.sync_copy(zvec_h, deg_local)
    ones16 = jnp.ones((16,), _f32)

    def _block(b, carry):
        pltpu.sync_copy(dst_h.at[wid, b], dst_v)

        def _step(i, c):
            idx = dst_v[pl.ds(i * 16, 16)]
            plsc.addupdate_scatter(deg_local, [idx], ones16)
            return c

        lax.fori_loop(0, DBLK // 16, _step, 0)
        return carry

    lax.fori_loop(0, DNIB, _block, 0)
    pltpu.sync_copy(deg_local, deg_out.at[wid])


def _wcomb_body(c0_ref, b0_ref, c1_ref, b1_ref, w0_ref, w1_ref):
    w0_ref[...] = jnp.dot(c0_ref[...], b0_ref[...],
                          preferred_element_type=_f32)
    w1_ref[...] = jnp.dot(c1_ref[...], b1_ref[...],
                          preferred_element_type=_f32)


def _combine_weights(coeffs0, bases0, coeffs1, bases1):
    w0, w1 = pl.pallas_call(
        _wcomb_body,
        out_shape=(jax.ShapeDtypeStruct((R, D * D), _f32),
                   jax.ShapeDtypeStruct((R, D * D), _f32)),
    )(coeffs0, bases0.reshape(NB, D * D), coeffs1, bases1.reshape(NB, D * D))
    return w0.reshape(R, D, D), w1.reshape(R, D, D)


def _prep_body(x_ref, w_ref, wself_ref, xw_ref, self_ref):
    x = x_ref[...]
    for r in range(R):
        xw_ref[r] = jnp.dot(x, w_ref[r], preferred_element_type=_f32)
    self_ref[...] = jnp.dot(x, wself_ref[...], preferred_element_type=_f32)


def _prep(x, w, wself):
    return pl.pallas_call(
        _prep_body,
        grid=(GRID,),
        in_specs=[
            pl.BlockSpec((BN, D), lambda i: (i, 0)),
            pl.BlockSpec((R, D, D), lambda i: (0, 0, 0)),
            pl.BlockSpec((D, D), lambda i: (0, 0)),
        ],
        out_specs=(
            pl.BlockSpec((R, BN, D), lambda i: (0, i, 0)),
            pl.BlockSpec((BN, D), lambda i: (i, 0)),
        ),
        out_shape=(jax.ShapeDtypeStruct((R, N, D), _f32),
                   jax.ShapeDtypeStruct((N, D), _f32)),
    )(x, w, wself)


def _mid_body(acc_ref, deg_ref, self_ref, w_ref, wself_ref,
              xw_ref, self1_ref):
    deg = jnp.sum(deg_ref[...], axis=1, keepdims=True)
    degc = jnp.maximum(deg, 1.0)
    x1 = acc_ref[...] / degc + self_ref[...]
    x1 = jnp.maximum(x1, 0.0)
    for r in range(R):
        xw_ref[r] = jnp.dot(x1, w_ref[r], preferred_element_type=_f32)
    self1_ref[...] = jnp.dot(x1, wself_ref[...], preferred_element_type=_f32)


def _mid(acc, degcol, self0, w, wself):
    return pl.pallas_call(
        _mid_body,
        grid=(GRID,),
        in_specs=[
            pl.BlockSpec((BN, D), lambda i: (i, 0)),
            pl.BlockSpec((BN, NW), lambda i: (i, 0)),
            pl.BlockSpec((BN, D), lambda i: (i, 0)),
            pl.BlockSpec((R, D, D), lambda i: (0, 0, 0)),
            pl.BlockSpec((D, D), lambda i: (0, 0)),
        ],
        out_specs=(
            pl.BlockSpec((R, BN, D), lambda i: (0, i, 0)),
            pl.BlockSpec((BN, D), lambda i: (i, 0)),
        ),
        out_shape=(jax.ShapeDtypeStruct((R, N, D), _f32),
                   jax.ShapeDtypeStruct((N, D), _f32)),
    )(acc, degcol, self0, w, wself)


def _post_body(acc_ref, deg_ref, self_ref, out_ref):
    deg = jnp.sum(deg_ref[...], axis=1, keepdims=True)
    degc = jnp.maximum(deg, 1.0)
    out_ref[...] = acc_ref[...] / degc + self_ref[...]


def _post(acc, degcol, self1):
    return pl.pallas_call(
        _post_body,
        grid=(GRID,),
        in_specs=[
            pl.BlockSpec((BN, D), lambda i: (i, 0)),
            pl.BlockSpec((BN, NW), lambda i: (i, 0)),
            pl.BlockSpec((BN, D), lambda i: (i, 0)),
        ],
        out_specs=pl.BlockSpec((BN, D), lambda i: (i, 0)),
        out_shape=jax.ShapeDtypeStruct((N, D), _f32),
    )(acc, degcol, self1)


def kernel(feature_list, adj_list, alignment_list, train_year,
           bases0, coeffs0, wself0, bases1, coeffs1, wself1):
    del alignment_list
    w0, w1 = _combine_weights(coeffs0, bases0, coeffs1, bases1)

    z64 = jnp.zeros((CH, D), _f32)
    zvec = jnp.zeros((NPAD,), _f32)

    pad = NW * EPW - E
    outs = []
    prev = None
    for t in range(T):
        x = feature_list[t]
        edges = adj_list[t].astype(jnp.int32)
        src, dst = edges[0], edges[1]
        rel = edges[2] % R
        gidx = rel * N + src
        gidxp = jnp.concatenate([gidx, jnp.zeros((pad,), jnp.int32)])
        dstp = jnp.concatenate([dst, jnp.full((pad,), N, jnp.int32)])
        gidx4 = gidxp.reshape(NW, NIB, IBLK, CH)
        dst4 = dstp.reshape(NW, NIB, IBLK, CH)
        dstd = dstp.reshape(NW, DNIB, DBLK)
        if prev is not None:
            gidx4, dst4, dstd, _ = lax.optimization_barrier(
                (gidx4, dst4, dstd, prev))

        deg = _sc_deg(dstd, zvec)
        degt = deg[:, :N].T
        xw0, s0 = _prep(x, w0, wself0)
        gidx4b, dst4b, _ = lax.optimization_barrier((gidx4, dst4, deg))
        acc0 = _sc_agg(xw0.reshape(R * N, D), gidx4b, dst4b, z64)
        xw1, s1 = _mid(acc0, degt, s0, w1, wself1)
        acc1 = _sc_agg(xw1.reshape(R * N, D), gidx4b, dst4b, z64)
        outs.append(_post(acc1, degt, s1))
        prev = acc1

    emb = jnp.stack(outs)
    mask = (jnp.arange(T) < train_year).astype(emb.dtype)
    return emb * mask[:, None, None]


# speedup vs baseline: 17.8667x; 2.0011x over previous
import functools

import jax
import jax.numpy as jnp
from jax import lax
from jax.experimental import pallas as pl
from jax.experimental.pallas import tpu as pltpu
from jax.experimental.pallas import tpu_sc as plsc

N = 10000
E = 320000
D = 128
R = 4
NB = 4
T = 2

NW = 16
NPAD = 10240
RPT = NPAD // NW
EPW = 20480
CH = 32
NSLOT = 4
IBLK = 64
NIB = EPW // (IBLK * CH)
DCH = 128
DIBLK = 32
DNIB = EPW // (DIBLK * DCH)
BN = 1000
GRID = N // BN

_f32 = jnp.float32
_i32 = jnp.int32

_sc_mesh = plsc.VectorSubcoreMesh(core_axis_name="c", subcore_axis_name="s",
                                  num_cores=1)
_sc_mesh2 = plsc.VectorSubcoreMesh(core_axis_name="c", subcore_axis_name="s",
                                   num_cores=2)

HALF = 5000
HPAD = 5248
PDUMP = 5003
RPT2 = HPAD // NW
PBLK = 1024
PNB = EPW // PBLK


@functools.partial(
    pl.kernel,
    out_type=jax.ShapeDtypeStruct((NPAD, D), _f32),
    mesh=_sc_mesh,
    scratch_types=[
        pltpu.VMEM((IBLK, CH), jnp.int32),
        pltpu.VMEM((IBLK, CH), jnp.int32),
        [pltpu.VMEM((CH, D), _f32)] * NSLOT,
        [pltpu.SemaphoreType.DMA] * NSLOT,
        [pltpu.SemaphoreType.DMA] * NSLOT,
        pltpu.VMEM_SHARED((NPAD, D), _f32),
    ],
)
def _sc_agg(table, gidx_h, dst_h, z_h,
            acc_out,
            gidx_v, dst_v, bufs, sem_g, sem_c, acc_sh):
    wid = lax.axis_index("s")

    pltpu.sync_copy(z_h, bufs[0])
    base = wid * RPT

    def _zero(i, carry):
        pltpu.sync_copy(bufs[0], acc_sh.at[pl.ds(base + i * CH, CH)])
        return carry

    lax.fori_loop(0, RPT // CH, _zero, 0)
    plsc.subcore_barrier()

    def _block(b, carry):
        pltpu.sync_copy(gidx_h.at[wid, b], gidx_v)
        pltpu.sync_copy(dst_h.at[wid, b], dst_v)
        for s in range(NSLOT):
            pltpu.async_copy(table.at[gidx_v.at[s]], bufs[s], sem_g[s])

        def _group(g, c):
            j0 = g * NSLOT
            for s in range(NSLOT):
                pltpu.make_async_copy(table.at[gidx_v.at[j0 + s]], bufs[s],
                                      sem_g[s]).wait()
                pltpu.async_copy(bufs[s], acc_sh.at[dst_v.at[j0 + s]],
                                 sem_c[s], add=True)
            for s in range(NSLOT):
                pltpu.make_async_copy(bufs[s], acc_sh.at[dst_v.at[j0 + s]],
                                      sem_c[s]).wait()

                @pl.when(g < IBLK // NSLOT - 1)
                def _prefetch(s=s, j0=j0):
                    pltpu.async_copy(table.at[gidx_v.at[j0 + NSLOT + s]],
                                     bufs[s], sem_g[s])

            return c

        lax.fori_loop(0, IBLK // NSLOT, _group, 0)
        return carry

    lax.fori_loop(0, NIB, _block, 0)
    plsc.subcore_barrier()

    pltpu.sync_copy(acc_sh.at[pl.ds(base, RPT)],
                    acc_out.at[pl.ds(base, RPT)])


@functools.partial(
    pl.kernel,
    out_type=(
        jax.ShapeDtypeStruct((2, NW, EPW), _i32),
        jax.ShapeDtypeStruct((2, NW, EPW), _i32),
    ),
    mesh=_sc_mesh,
    compiler_params=pltpu.CompilerParams(needs_layout_passes=False),
    scratch_types=[
        pltpu.VMEM((PBLK,), _i32),
        pltpu.VMEM((PBLK,), _i32),
        pltpu.VMEM((EPW,), _i32),
        pltpu.VMEM((EPW,), _i32),
        pltpu.VMEM((EPW,), _i32),
        pltpu.VMEM((EPW,), _i32),
    ],
)
def _sc_part(gidx_h, dst_h, fillg_h, filld_h,
             gout, dout,
             g_blk, d_blk, lo_g, lo_d, hi_g, hi_d):
    wid = lax.axis_index("s")
    pltpu.sync_copy(fillg_h, lo_g)
    pltpu.sync_copy(fillg_h, hi_g)
    pltpu.sync_copy(filld_h, lo_d)
    pltpu.sync_copy(filld_h, hi_d)

    def _block(b, ptrs):
        pltpu.sync_copy(gidx_h.at[wid, b], g_blk)
        pltpu.sync_copy(dst_h.at[wid, b], d_blk)

        def _vec(i, p):
            plo, phi = p
            gv = g_blk[pl.ds(i * 16, 16)]
            dv = d_blk[pl.ds(i * 16, 16)]
            mlo = dv < HALF
            mhi = jnp.logical_and(dv >= HALF, dv < N)
            plsc.store_compressed(lo_g.at[pl.ds(plo, 16)], gv, mask=mlo)
            plsc.store_compressed(lo_d.at[pl.ds(plo, 16)], dv, mask=mlo)
            plsc.store_compressed(hi_g.at[pl.ds(phi, 16)], gv, mask=mhi)
            plsc.store_compressed(hi_d.at[pl.ds(phi, 16)], dv - HALF, mask=mhi)
            plo = plo + jnp.sum(mlo.astype(_i32))
            phi = phi + jnp.sum(mhi.astype(_i32))
            return (plo, phi)

        return lax.fori_loop(0, PBLK // 16, _vec, ptrs)

    lax.fori_loop(0, PNB, _block, (jnp.int32(0), jnp.int32(0)))
    pltpu.sync_copy(lo_g, gout.at[0, wid])
    pltpu.sync_copy(lo_d, dout.at[0, wid])
    pltpu.sync_copy(hi_g, gout.at[1, wid])
    pltpu.sync_copy(hi_d, dout.at[1, wid])


NSLOT2 = 2
ICH2 = PBLK // CH


@functools.partial(
    pl.kernel,
    out_type=jax.ShapeDtypeStruct((2, HPAD, D), _f32),
    mesh=_sc_mesh2,
    compiler_params=pltpu.CompilerParams(needs_layout_passes=False),
    scratch_types=[
        pltpu.VMEM((ICH2, CH), _i32),
        pltpu.VMEM((ICH2, CH), _i32),
        [pltpu.VMEM((CH, D), _f32)] * NSLOT2,
        [pltpu.SemaphoreType.DMA] * NSLOT2,
        [pltpu.SemaphoreType.DMA] * NSLOT2,
        pltpu.VMEM_SHARED((HPAD, D), _f32),
    ],
)
def _sc_agg2(table, gidx_h, dst_h, z_h,
             acc_out,
             gidx_v, dst_v, bufs, sem_g, sem_c, acc_sh):
    cid = lax.axis_index("c")
    wid = lax.axis_index("s")

    pltpu.sync_copy(z_h, bufs[0])
    base = wid * RPT2

    def _zero(i, carry):
        pltpu.sync_copy(bufs[0], acc_sh.at[pl.ds(base + i * CH, CH)])
        return carry

    lax.fori_loop(0, RPT2 // CH, _zero, 0)
    pltpu.sync_copy(bufs[0].at[pl.ds(0, RPT2 - (RPT2 // CH) * CH)],
                    acc_sh.at[pl.ds(base + (RPT2 // CH) * CH,
                                    RPT2 - (RPT2 // CH) * CH)])
    plsc.subcore_barrier()

    iota16 = lax.iota(_i32, 16)
    zeros16 = iota16 * 0

    def _block(b, carry):
        pltpu.sync_copy(gidx_h.at[cid, wid, b], gidx_v)
        pltpu.sync_copy(dst_h.at[cid, wid, b], dst_v)
        f1 = plsc.load_gather(dst_v, [iota16, zeros16])
        f2 = plsc.load_gather(dst_v, [iota16 + 16, zeros16])
        nch = (jnp.sum((f1 < HALF).astype(_i32))
               + jnp.sum((f2 < HALF).astype(_i32)))
        ngrp = (nch + NSLOT2 - 1) // NSLOT2

        @pl.when(ngrp > 0)
        def _prime():
            for s in range(NSLOT2):
                pltpu.async_copy(table.at[gidx_v.at[s]], bufs[s], sem_g[s])

        def _group(g, c):
            j0 = g * NSLOT2
            for s in range(NSLOT2):
                pltpu.make_async_copy(table.at[gidx_v.at[j0 + s]],
                                      bufs[s], sem_g[s]).wait()
                pltpu.async_copy(bufs[s], acc_sh.at[dst_v.at[j0 + s]],
                                 sem_c[s], add=True)
            for s in range(NSLOT2):
                pltpu.make_async_copy(bufs[s], acc_sh.at[dst_v.at[j0 + s]],
                                      sem_c[s]).wait()

                @pl.when(g < ngrp - 1)
                def _prefetch(s=s, j0=j0):
                    pltpu.async_copy(
                        table.at[gidx_v.at[j0 + NSLOT2 + s]],
                        bufs[s], sem_g[s])

            return c

        lax.fori_loop(0, ngrp, _group, 0)
        return carry

    lax.fori_loop(0, PNB, _block, 0)
    plsc.subcore_barrier()
    pltpu.sync_copy(acc_sh.at[pl.ds(base, RPT2)],
                    acc_out.at[cid, pl.ds(base, RPT2)])


DBLK = DIBLK * DCH


@functools.partial(
    pl.kernel,
    out_type=jax.ShapeDtypeStruct((NW, NPAD), _f32),
    mesh=_sc_mesh,
    compiler_params=pltpu.CompilerParams(needs_layout_passes=False),
    scratch_types=[
        pltpu.VMEM((DBLK,), jnp.int32),
        pltpu.VMEM((NPAD,), _f32),
    ],
)
def _sc_deg(dst_h, zvec_h, deg_out, dst_v, deg_local):
    wid = lax.axis_index("s")
    pltpu.sync_copy(zvec_h, deg_local)
    ones16 = jnp.ones((16,), _f32)

    def _block(b, carry):
        pltpu.sync_copy(dst_h.at[wid, b], dst_v)

        def _step(i, c):
            idx = dst_v[pl.ds(i * 16, 16)]
            plsc.addupdate_scatter(deg_local, [idx], ones16)
            return c

        lax.fori_loop(0, DBLK // 16, _step, 0)
        return carry

    lax.fori_loop(0, DNIB, _block, 0)
    pltpu.sync_copy(deg_local, deg_out.at[wid])


def _wcomb_body(c0_ref, b0_ref, c1_ref, b1_ref, w0_ref, w1_ref):
    w0_ref[...] = jnp.dot(c0_ref[...], b0_ref[...],
                          preferred_element_type=_f32)
    w1_ref[...] = jnp.dot(c1_ref[...], b1_ref[...],
                          preferred_element_type=_f32)


def _combine_weights(coeffs0, bases0, coeffs1, bases1):
    w0, w1 = pl.pallas_call(
        _wcomb_body,
        out_shape=(jax.ShapeDtypeStruct((R, D * D), _f32),
                   jax.ShapeDtypeStruct((R, D * D), _f32)),
    )(coeffs0, bases0.reshape(NB, D * D), coeffs1, bases1.reshape(NB, D * D))
    return w0.reshape(R, D, D), w1.reshape(R, D, D)


def _prep_body(x_ref, w_ref, wself_ref, xw_ref, self_ref):
    x = x_ref[...]
    for r in range(R):
        xw_ref[r] = jnp.dot(x, w_ref[r], preferred_element_type=_f32)
    self_ref[...] = jnp.dot(x, wself_ref[...], preferred_element_type=_f32)


def _prep(x, w, wself):
    return pl.pallas_call(
        _prep_body,
        grid=(GRID,),
        in_specs=[
            pl.BlockSpec((BN, D), lambda i: (i, 0)),
            pl.BlockSpec((R, D, D), lambda i: (0, 0, 0)),
            pl.BlockSpec((D, D), lambda i: (0, 0)),
        ],
        out_specs=(
            pl.BlockSpec((R, BN, D), lambda i: (0, i, 0)),
            pl.BlockSpec((BN, D), lambda i: (i, 0)),
        ),
        out_shape=(jax.ShapeDtypeStruct((R, N, D), _f32),
                   jax.ShapeDtypeStruct((N, D), _f32)),
    )(x, w, wself)


def _acc_spec():
    per_half = HALF // BN
    return pl.BlockSpec((1, BN, D),
                        lambda i: (i // per_half, i % per_half, 0))


def _mid_body(acc_ref, deg_ref, self_ref, w_ref, wself_ref,
              xw_ref, self1_ref):
    deg = jnp.sum(deg_ref[...], axis=1, keepdims=True)
    degc = jnp.maximum(deg, 1.0)
    x1 = acc_ref[0] / degc + self_ref[...]
    x1 = jnp.maximum(x1, 0.0)
    for r in range(R):
        xw_ref[r] = jnp.dot(x1, w_ref[r], preferred_element_type=_f32)
    self1_ref[...] = jnp.dot(x1, wself_ref[...], preferred_element_type=_f32)


def _mid(acc, degcol, self0, w, wself):
    return pl.pallas_call(
        _mid_body,
        grid=(GRID,),
        in_specs=[
            _acc_spec(),
            pl.BlockSpec((BN, NW), lambda i: (i, 0)),
            pl.BlockSpec((BN, D), lambda i: (i, 0)),
            pl.BlockSpec((R, D, D), lambda i: (0, 0, 0)),
            pl.BlockSpec((D, D), lambda i: (0, 0)),
        ],
        out_specs=(
            pl.BlockSpec((R, BN, D), lambda i: (0, i, 0)),
            pl.BlockSpec((BN, D), lambda i: (i, 0)),
        ),
        out_shape=(jax.ShapeDtypeStruct((R, N, D), _f32),
                   jax.ShapeDtypeStruct((N, D), _f32)),
    )(acc, degcol, self0, w, wself)


def _post_body(acc_ref, deg_ref, self_ref, out_ref):
    deg = jnp.sum(deg_ref[...], axis=1, keepdims=True)
    degc = jnp.maximum(deg, 1.0)
    out_ref[...] = acc_ref[0] / degc + self_ref[...]


def _post(acc, degcol, self1):
    return pl.pallas_call(
        _post_body,
        grid=(GRID,),
        in_specs=[
            _acc_spec(),
            pl.BlockSpec((BN, NW), lambda i: (i, 0)),
            pl.BlockSpec((BN, D), lambda i: (i, 0)),
        ],
        out_specs=pl.BlockSpec((BN, D), lambda i: (i, 0)),
        out_shape=jax.ShapeDtypeStruct((N, D), _f32),
    )(acc, degcol, self1)


def kernel(feature_list, adj_list, alignment_list, train_year,
           bases0, coeffs0, wself0, bases1, coeffs1, wself1):
    del alignment_list
    w0, w1 = _combine_weights(coeffs0, bases0, coeffs1, bases1)

    z64 = jnp.zeros((CH, D), _f32)
    zvec = jnp.zeros((NPAD,), _f32)
    fillg = jnp.zeros((EPW,), _i32)
    filld = jnp.full((EPW,), PDUMP, _i32)

    pad = NW * EPW - E
    outs = []
    prev = None
    for t in range(T):
        x = feature_list[t]
        edges = adj_list[t].astype(jnp.int32)
        src, dst = edges[0], edges[1]
        rel = edges[2] % R
        gidx = rel * N + src
        gidxp = jnp.concatenate([gidx, jnp.zeros((pad,), jnp.int32)])
        dstp = jnp.concatenate([dst, jnp.full((pad,), N, jnp.int32)])
        gp = gidxp.reshape(NW, PNB, PBLK)
        dp = dstp.reshape(NW, PNB, PBLK)
        dstd = dstp.reshape(NW, DNIB, DBLK)
        if prev is not None:
            gp, dp, dstd, _ = lax.optimization_barrier((gp, dp, dstd, prev))

        pg, pd = _sc_part(gp, dp, fillg, filld)
        dstd_b, _ = lax.optimization_barrier((dstd, pg))
        deg = _sc_deg(dstd_b, zvec)
        degt = deg[:, :N].T
        xw0, s0 = _prep(x, w0, wself0)
        pg4 = pg.reshape(2, NW, PNB, ICH2, CH)
        pd4 = pd.reshape(2, NW, PNB, ICH2, CH)
        pg4, pd4, _ = lax.optimization_barrier((pg4, pd4, deg))
        acc0 = _sc_agg2(xw0.reshape(R * N, D), pg4, pd4, z64)
        xw1, s1 = _mid(acc0, degt, s0, w1, wself1)
        acc1 = _sc_agg2(xw1.reshape(R * N, D), pg4, pd4, z64)
        outs.append(_post(acc1, degt, s1))
        prev = acc1

    emb = jnp.stack(outs)
    mask = (jnp.arange(T) < train_year).astype(emb.dtype)
    return emb * mask[:, None, None]


# speedup vs baseline: 18.1371x; 1.0151x over previous
import functools

import jax
import jax.numpy as jnp
from jax import lax
from jax.experimental import pallas as pl
from jax.experimental.pallas import tpu as pltpu
from jax.experimental.pallas import tpu_sc as plsc

N = 10000
E = 320000
D = 128
R = 4
NB = 4
T = 2

NW = 16
NPAD = 10240
RPT = NPAD // NW
EPW = 20480
CH = 32
NSLOT = 4
IBLK = 64
NIB = EPW // (IBLK * CH)
DCH = 128
DIBLK = 32
DNIB = EPW // (DIBLK * DCH)
BN = 1000
GRID = N // BN

_f32 = jnp.float32
_i32 = jnp.int32

_sc_mesh = plsc.VectorSubcoreMesh(core_axis_name="c", subcore_axis_name="s",
                                  num_cores=1)
_sc_mesh2 = plsc.VectorSubcoreMesh(core_axis_name="c", subcore_axis_name="s",
                                   num_cores=2)

HALF = 5000
HPAD = 5248
PDUMP = 5003
RPT2 = HPAD // NW
PBLK = 1024
PNB = EPW // PBLK


@functools.partial(
    pl.kernel,
    out_type=jax.ShapeDtypeStruct((NPAD, D), _f32),
    mesh=_sc_mesh,
    scratch_types=[
        pltpu.VMEM((IBLK, CH), jnp.int32),
        pltpu.VMEM((IBLK, CH), jnp.int32),
        [pltpu.VMEM((CH, D), _f32)] * NSLOT,
        [pltpu.SemaphoreType.DMA] * NSLOT,
        [pltpu.SemaphoreType.DMA] * NSLOT,
        pltpu.VMEM_SHARED((NPAD, D), _f32),
    ],
)
def _sc_agg(table, gidx_h, dst_h, z_h,
            acc_out,
            gidx_v, dst_v, bufs, sem_g, sem_c, acc_sh):
    wid = lax.axis_index("s")

    pltpu.sync_copy(z_h, bufs[0])
    base = wid * RPT

    def _zero(i, carry):
        pltpu.sync_copy(bufs[0], acc_sh.at[pl.ds(base + i * CH, CH)])
        return carry

    lax.fori_loop(0, RPT // CH, _zero, 0)
    plsc.subcore_barrier()

    def _block(b, carry):
        pltpu.sync_copy(gidx_h.at[wid, b], gidx_v)
        pltpu.sync_copy(dst_h.at[wid, b], dst_v)
        for s in range(NSLOT):
            pltpu.async_copy(table.at[gidx_v.at[s]], bufs[s], sem_g[s])

        def _group(g, c):
            j0 = g * NSLOT
            for s in range(NSLOT):
                pltpu.make_async_copy(table.at[gidx_v.at[j0 + s]], bufs[s],
                                      sem_g[s]).wait()
                pltpu.async_copy(bufs[s], acc_sh.at[dst_v.at[j0 + s]],
                                 sem_c[s], add=True)
            for s in range(NSLOT):
                pltpu.make_async_copy(bufs[s], acc_sh.at[dst_v.at[j0 + s]],
                                      sem_c[s]).wait()

                @pl.when(g < IBLK // NSLOT - 1)
                def _prefetch(s=s, j0=j0):
                    pltpu.async_copy(table.at[gidx_v.at[j0 + NSLOT + s]],
                                     bufs[s], sem_g[s])

            return c

        lax.fori_loop(0, IBLK // NSLOT, _group, 0)
        return carry

    lax.fori_loop(0, NIB, _block, 0)
    plsc.subcore_barrier()

    pltpu.sync_copy(acc_sh.at[pl.ds(base, RPT)],
                    acc_out.at[pl.ds(base, RPT)])


@functools.partial(
    pl.kernel,
    out_type=(
        jax.ShapeDtypeStruct((2, NW, EPW), _i32),
        jax.ShapeDtypeStruct((2, NW, EPW), _i32),
        jax.ShapeDtypeStruct((NW, NPAD), _f32),
    ),
    mesh=_sc_mesh,
    compiler_params=pltpu.CompilerParams(needs_layout_passes=False),
    scratch_types=[
        pltpu.VMEM((PBLK,), _i32),
        pltpu.VMEM((PBLK,), _i32),
        pltpu.VMEM((EPW,), _i32),
        pltpu.VMEM((EPW,), _i32),
        pltpu.VMEM((EPW,), _i32),
        pltpu.VMEM((EPW,), _i32),
        pltpu.VMEM((NPAD,), _f32),
    ],
)
def _sc_part(gidx_h, dst_h, fillg_h, filld_h, zvec_h,
             gout, dout, deg_out,
             g_blk, d_blk, lo_g, lo_d, hi_g, hi_d, deg_local):
    wid = lax.axis_index("s")
    pltpu.sync_copy(fillg_h, lo_g)
    pltpu.sync_copy(fillg_h, hi_g)
    pltpu.sync_copy(filld_h, lo_d)
    pltpu.sync_copy(filld_h, hi_d)
    pltpu.sync_copy(zvec_h, deg_local)
    ones16 = jnp.ones((16,), _f32)

    def _block(b, ptrs):
        pltpu.sync_copy(gidx_h.at[wid, b], g_blk)
        pltpu.sync_copy(dst_h.at[wid, b], d_blk)

        def _vec(i, p):
            plo, phi = p
            gv = g_blk[pl.ds(i * 16, 16)]
            dv = d_blk[pl.ds(i * 16, 16)]
            mlo = dv < HALF
            mhi = jnp.logical_and(dv >= HALF, dv < N)
            plsc.store_compressed(lo_g.at[pl.ds(plo, 16)], gv, mask=mlo)
            plsc.store_compressed(lo_d.at[pl.ds(plo, 16)], dv, mask=mlo)
            plsc.store_compressed(hi_g.at[pl.ds(phi, 16)], gv, mask=mhi)
            plsc.store_compressed(hi_d.at[pl.ds(phi, 16)], dv - HALF, mask=mhi)
            plsc.addupdate_scatter(deg_local, [dv], ones16)
            plo = plo + jnp.sum(mlo.astype(_i32))
            phi = phi + jnp.sum(mhi.astype(_i32))
            return (plo, phi)

        return lax.fori_loop(0, PBLK // 16, _vec, ptrs)

    lax.fori_loop(0, PNB, _block, (jnp.int32(0), jnp.int32(0)))
    pltpu.sync_copy(lo_g, gout.at[0, wid])
    pltpu.sync_copy(lo_d, dout.at[0, wid])
    pltpu.sync_copy(hi_g, gout.at[1, wid])
    pltpu.sync_copy(hi_d, dout.at[1, wid])
    pltpu.sync_copy(deg_local, deg_out.at[wid])


NSLOT2 = 2
ICH2 = PBLK // CH


@functools.partial(
    pl.kernel,
    out_type=jax.ShapeDtypeStruct((2, HPAD, D), _f32),
    mesh=_sc_mesh2,
    compiler_params=pltpu.CompilerParams(needs_layout_passes=False),
    scratch_types=[
        pltpu.VMEM((ICH2, CH), _i32),
        pltpu.VMEM((ICH2, CH), _i32),
        [pltpu.VMEM((CH, D), _f32)] * NSLOT2,
        [pltpu.SemaphoreType.DMA] * NSLOT2,
        [pltpu.SemaphoreType.DMA] * NSLOT2,
        pltpu.VMEM_SHARED((HPAD, D), _f32),
    ],
)
def _sc_agg2(table, gidx_h, dst_h, z_h,
             acc_out,
             gidx_v, dst_v, bufs, sem_g, sem_c, acc_sh):
    cid = lax.axis_index("c")
    wid = lax.axis_index("s")

    pltpu.sync_copy(z_h, bufs[0])
    base = wid * RPT2

    def _zero(i, carry):
        pltpu.sync_copy(bufs[0], acc_sh.at[pl.ds(base + i * CH, CH)])
        return carry

    lax.fori_loop(0, RPT2 // CH, _zero, 0)
    pltpu.sync_copy(bufs[0].at[pl.ds(0, RPT2 - (RPT2 // CH) * CH)],
                    acc_sh.at[pl.ds(base + (RPT2 // CH) * CH,
                                    RPT2 - (RPT2 // CH) * CH)])
    plsc.subcore_barrier()

    iota16 = lax.iota(_i32, 16)
    zeros16 = iota16 * 0

    def _block(b, carry):
        pltpu.sync_copy(gidx_h.at[cid, wid, b], gidx_v)
        pltpu.sync_copy(dst_h.at[cid, wid, b], dst_v)
        f1 = plsc.load_gather(dst_v, [iota16, zeros16])
        f2 = plsc.load_gather(dst_v, [iota16 + 16, zeros16])
        nch = (jnp.sum((f1 < HALF).astype(_i32))
               + jnp.sum((f2 < HALF).astype(_i32)))
        ngrp = (nch + NSLOT2 - 1) // NSLOT2

        @pl.when(ngrp > 0)
        def _prime():
            for s in range(NSLOT2):
                pltpu.async_copy(table.at[gidx_v.at[s]], bufs[s], sem_g[s])

        def _group(g, c):
            j0 = g * NSLOT2
            for s in range(NSLOT2):
                pltpu.make_async_copy(table.at[gidx_v.at[j0 + s]],
                                      bufs[s], sem_g[s]).wait()
                pltpu.async_copy(bufs[s], acc_sh.at[dst_v.at[j0 + s]],
                                 sem_c[s], add=True)
            for s in range(NSLOT2):
                pltpu.make_async_copy(bufs[s], acc_sh.at[dst_v.at[j0 + s]],
                                      sem_c[s]).wait()

                @pl.when(g < ngrp - 1)
                def _prefetch(s=s, j0=j0):
                    pltpu.async_copy(
                        table.at[gidx_v.at[j0 + NSLOT2 + s]],
                        bufs[s], sem_g[s])

            return c

        lax.fori_loop(0, ngrp, _group, 0)
        return carry

    lax.fori_loop(0, PNB, _block, 0)
    plsc.subcore_barrier()
    pltpu.sync_copy(acc_sh.at[pl.ds(base, RPT2)],
                    acc_out.at[cid, pl.ds(base, RPT2)])


DBLK = DIBLK * DCH


@functools.partial(
    pl.kernel,
    out_type=jax.ShapeDtypeStruct((NW, NPAD), _f32),
    mesh=_sc_mesh,
    compiler_params=pltpu.CompilerParams(needs_layout_passes=False),
    scratch_types=[
        pltpu.VMEM((DBLK,), jnp.int32),
        pltpu.VMEM((NPAD,), _f32),
    ],
)
def _sc_deg(dst_h, zvec_h, deg_out, dst_v, deg_local):
    wid = lax.axis_index("s")
    pltpu.sync_copy(zvec_h, deg_local)
    ones16 = jnp.ones((16,), _f32)

    def _block(b, carry):
        pltpu.sync_copy(dst_h.at[wid, b], dst_v)

        def _step(i, c):
            idx = dst_v[pl.ds(i * 16, 16)]
            plsc.addupdate_scatter(deg_local, [idx], ones16)
            return c

        lax.fori_loop(0, DBLK // 16, _step, 0)
        return carry

    lax.fori_loop(0, DNIB, _block, 0)
    pltpu.sync_copy(deg_local, deg_out.at[wid])


def _wcomb_body(c0_ref, b0_ref, c1_ref, b1_ref, w0_ref, w1_ref):
    w0_ref[...] = jnp.dot(c0_ref[...], b0_ref[...],
                          preferred_element_type=_f32)
    w1_ref[...] = jnp.dot(c1_ref[...], b1_ref[...],
                          preferred_element_type=_f32)


def _combine_weights(coeffs0, bases0, coeffs1, bases1):
    w0, w1 = pl.pallas_call(
        _wcomb_body,
        out_shape=(jax.ShapeDtypeStruct((R, D * D), _f32),
                   jax.ShapeDtypeStruct((R, D * D), _f32)),
    )(coeffs0, bases0.reshape(NB, D * D), coeffs1, bases1.reshape(NB, D * D))
    return w0.reshape(R, D, D), w1.reshape(R, D, D)


def _prep_body(x_ref, w_ref, wself_ref, xw_ref, self_ref):
    x = x_ref[...]
    for r in range(R):
        xw_ref[r] = jnp.dot(x, w_ref[r], preferred_element_type=_f32)
    self_ref[...] = jnp.dot(x, wself_ref[...], preferred_element_type=_f32)


def _prep(x, w, wself):
    return pl.pallas_call(
        _prep_body,
        grid=(GRID,),
        in_specs=[
            pl.BlockSpec((BN, D), lambda i: (i, 0)),
            pl.BlockSpec((R, D, D), lambda i: (0, 0, 0)),
            pl.BlockSpec((D, D), lambda i: (0, 0)),
        ],
        out_specs=(
            pl.BlockSpec((R, BN, D), lambda i: (0, i, 0)),
            pl.BlockSpec((BN, D), lambda i: (i, 0)),
        ),
        out_shape=(jax.ShapeDtypeStruct((R, N, D), _f32),
                   jax.ShapeDtypeStruct((N, D), _f32)),
    )(x, w, wself)


def _acc_spec():
    per_half = HALF // BN
    return pl.BlockSpec((1, BN, D),
                        lambda i: (i // per_half, i % per_half, 0))


def _mid_body(acc_ref, deg_ref, self_ref, w_ref, wself_ref,
              xw_ref, self1_ref):
    deg = jnp.sum(deg_ref[...], axis=1, keepdims=True)
    degc = jnp.maximum(deg, 1.0)
    x1 = acc_ref[0] / degc + self_ref[...]
    x1 = jnp.maximum(x1, 0.0)
    for r in range(R):
        xw_ref[r] = jnp.dot(x1, w_ref[r], preferred_element_type=_f32)
    self1_ref[...] = jnp.dot(x1, wself_ref[...], preferred_element_type=_f32)


def _mid(acc, degcol, self0, w, wself):
    return pl.pallas_call(
        _mid_body,
        grid=(GRID,),
        in_specs=[
            _acc_spec(),
            pl.BlockSpec((BN, NW), lambda i: (i, 0)),
            pl.BlockSpec((BN, D), lambda i: (i, 0)),
            pl.BlockSpec((R, D, D), lambda i: (0, 0, 0)),
            pl.BlockSpec((D, D), lambda i: (0, 0)),
        ],
        out_specs=(
            pl.BlockSpec((R, BN, D), lambda i: (0, i, 0)),
            pl.BlockSpec((BN, D), lambda i: (i, 0)),
        ),
        out_shape=(jax.ShapeDtypeStruct((R, N, D), _f32),
                   jax.ShapeDtypeStruct((N, D), _f32)),
    )(acc, degcol, self0, w, wself)


def _post_body(acc_ref, deg_ref, self_ref, out_ref):
    deg = jnp.sum(deg_ref[...], axis=1, keepdims=True)
    degc = jnp.maximum(deg, 1.0)
    out_ref[...] = acc_ref[0] / degc + self_ref[...]


def _post(acc, degcol, self1):
    return pl.pallas_call(
        _post_body,
        grid=(GRID,),
        in_specs=[
            _acc_spec(),
            pl.BlockSpec((BN, NW), lambda i: (i, 0)),
            pl.BlockSpec((BN, D), lambda i: (i, 0)),
        ],
        out_specs=pl.BlockSpec((BN, D), lambda i: (i, 0)),
        out_shape=jax.ShapeDtypeStruct((N, D), _f32),
    )(acc, degcol, self1)


def kernel(feature_list, adj_list, alignment_list, train_year,
           bases0, coeffs0, wself0, bases1, coeffs1, wself1):
    del alignment_list
    w0, w1 = _combine_weights(coeffs0, bases0, coeffs1, bases1)

    z64 = jnp.zeros((CH, D), _f32)
    zvec = jnp.zeros((NPAD,), _f32)
    fillg = jnp.zeros((EPW,), _i32)
    filld = jnp.full((EPW,), PDUMP, _i32)

    pad = NW * EPW - E
    outs = []
    prev = None
    for t in range(T):
        x = feature_list[t]
        edges = adj_list[t].astype(jnp.int32)
        src, dst = edges[0], edges[1]
        rel = edges[2] % R
        gidx = rel * N + src
        gidxp = jnp.concatenate([gidx, jnp.zeros((pad,), jnp.int32)])
        dstp = jnp.concatenate([dst, jnp.full((pad,), N, jnp.int32)])
        gp = gidxp.reshape(NW, PNB, PBLK)
        dp = dstp.reshape(NW, PNB, PBLK)
        if prev is not None:
            gp, dp, _ = lax.optimization_barrier((gp, dp, prev))

        pg, pd, deg = _sc_part(gp, dp, fillg, filld, zvec)
        degt = deg[:, :N].T
        xw0, s0 = _prep(x, w0, wself0)
        pg4 = pg.reshape(2, NW, PNB, ICH2, CH)
        pd4 = pd.reshape(2, NW, PNB, ICH2, CH)
        acc0 = _sc_agg2(xw0.reshape(R * N, D), pg4, pd4, z64)
        xw1, s1 = _mid(acc0, degt, s0, w1, wself1)
        acc1 = _sc_agg2(xw1.reshape(R * N, D), pg4, pd4, z64)
        outs.append(_post(acc1, degt, s1))
        prev = acc1

    emb = jnp.stack(outs)
    mask = (jnp.arange(T) < train_year).astype(emb.dtype)
    return emb * mask[:, None, None]
